# baseline (speedup 1.0000x reference)
import functools

import jax
import jax.numpy as jnp
import numpy as np
from jax import lax
from jax.experimental import pallas as pl
from jax.experimental.pallas import tpu as pltpu
from jax.experimental.pallas import tpu_sc as plsc

N = 10000
E = 320000
D = 128
G = 100
NPG = 100

NPAD = 10064
EPAD = 327680
CHUNKS = 80
CW = 128
IBLK = 16
DW = 16
DEGP = 10240

_MESH = plsc.VectorSubcoreMesh(core_axis_name="c", subcore_axis_name="s",
                               num_cores=2, num_subcores=16)


@functools.partial(
    pl.kernel,
    out_type=jax.ShapeDtypeStruct((2, DEGP, DW), jnp.float32),
    mesh=_MESH,
    compiler_params=pltpu.CompilerParams(use_tc_tiling_on_sc=False),
    scratch_types=[
        pltpu.MemorySpace.VMEM_SHARED((DEGP, DW), jnp.float32),
        pltpu.MemorySpace.VMEM((CHUNKS, CW), jnp.int32),
        pltpu.MemorySpace.VMEM((CW, DW), jnp.float32),
        pltpu.MemorySpace.VMEM((640, DW), jnp.float32),
    ],
)
def _deg_sc(dstr_hbm, ones_hbm, z1_hbm, out_hbm, dacc, dst_v, ones_v, z_v):
    c = lax.axis_index("c")
    s = lax.axis_index("s")
    wid = s * 2 + c
    pltpu.sync_copy(z1_hbm, z_v)
    pltpu.sync_copy(z_v, dacc.at[pl.ds(s * 640, 640)])
    pltpu.sync_copy(ones_hbm, ones_v)
    pltpu.sync_copy(dstr_hbm.at[wid], dst_v)
    plsc.subcore_barrier()

    def chunk(j, carry):
        pltpu.sync_copy(ones_v, dacc.at[dst_v.at[j]], add=True)
        return carry
    lax.fori_loop(0, CHUNKS, chunk, 0)
    plsc.subcore_barrier()
    pltpu.sync_copy(dacc.at[pl.ds(s * 640, 640)],
                    out_hbm.at[c, pl.ds(s * 640, 640)])


@functools.partial(
    pl.kernel,
    out_type=jax.ShapeDtypeStruct((2, N, D), jnp.float32),
    mesh=_MESH,
    compiler_params=pltpu.CompilerParams(use_tc_tiling_on_sc=False),
    scratch_types=[
        pltpu.MemorySpace.VMEM_SHARED((NPAD, D), jnp.float32),
        pltpu.MemorySpace.VMEM((IBLK, CW), jnp.int32),
        pltpu.MemorySpace.VMEM((IBLK, CW), jnp.int32),
        pltpu.MemorySpace.VMEM((CW, D), jnp.float32),
        pltpu.MemorySpace.VMEM((CW, D), jnp.float32),
        pltpu.SemaphoreType.DMA,
        pltpu.SemaphoreType.DMA,
        pltpu.SemaphoreType.DMA,
        pltpu.SemaphoreType.DMA,
    ],
)
def _scatter_sc(xs_hbm, srcr_hbm, dstr_hbm, out_hbm,
                acc_s, src_v, dst_v, rows0, rows1, g0, g1, s0, s1):
    c = lax.axis_index("c")
    s = lax.axis_index("s")
    wid = s * 2 + c
    for j in range(8):
        i = s + 16 * j

        @pl.when(i < 125)
        def _():
            r0 = 80 * i
            pltpu.sync_copy(xs_hbm.at[pl.ds(r0, 80)], rows0.at[pl.ds(0, 80)])
            pltpu.sync_copy(rows0.at[pl.ds(0, 80)], acc_s.at[pl.ds(r0, 80)])
    plsc.subcore_barrier()

    def wait_g(rows, sem):
        pltpu.make_async_copy(xs_hbm.at[src_v.at[0]], rows, sem).wait()

    def wait_s(rows, sem):
        pltpu.make_async_copy(rows, acc_s.at[dst_v.at[0]], sem).wait()

    def oblk(k, carry):
        pltpu.sync_copy(srcr_hbm.at[wid, pl.ds(IBLK * k, IBLK)], src_v)
        pltpu.sync_copy(dstr_hbm.at[wid, pl.ds(IBLK * k, IBLK)], dst_v)
        pltpu.async_copy(xs_hbm.at[src_v.at[0]], rows0, g0)

        def pair(p, carry2):
            j = 2 * p
            wait_g(rows0, g0)
            pltpu.async_copy(rows0, acc_s.at[dst_v.at[j]], s0, add=True)

            @pl.when(p > 0)
            def _():
                wait_s(rows1, s1)
            pltpu.async_copy(xs_hbm.at[src_v.at[j + 1]], rows1, g1)
            wait_g(rows1, g1)
            pltpu.async_copy(rows1, acc_s.at[dst_v.at[j + 1]], s1, add=True)
            wait_s(rows0, s0)

            @pl.when(j + 2 < IBLK)
            def _():
                pltpu.async_copy(xs_hbm.at[src_v.at[j + 2]], rows0, g0)
            return carry2
        carry = lax.fori_loop(0, IBLK // 2, pair, carry)
        wait_s(rows1, s1)
        return carry
    lax.fori_loop(0, CHUNKS // IBLK, oblk, 0)
    plsc.subcore_barrier()

    for j in range(8):
        i = s + 16 * j

        @pl.when(i < 125)
        def _():
            r0 = 80 * i
            pltpu.sync_copy(acc_s.at[pl.ds(r0, 80)], rows0.at[pl.ds(0, 80)])
            pltpu.sync_copy(rows0.at[pl.ds(0, 80)], out_hbm.at[c, pl.ds(r0, 80)])


_ROWS_B = 400


def _pre_tc_body(x_ref, deg_ref, w_ref, xs_ref):
    deg = deg_ref[...]
    dinv = lax.rsqrt(deg[0, :, 0:1] + deg[1, :, 0:1] + 1.0)
    xs_ref[...] = dinv * jnp.dot(x_ref[...], w_ref[...],
                                 preferred_element_type=jnp.float32)


def _pre_tc(x, deg2, w0):
    return pl.pallas_call(
        _pre_tc_body,
        grid=(N // _ROWS_B,),
        in_specs=[
            pl.BlockSpec((_ROWS_B, D), lambda i: (i, 0)),
            pl.BlockSpec((2, _ROWS_B, DW), lambda i: (0, i, 0)),
            pl.BlockSpec((D, D), lambda i: (0, 0)),
        ],
        out_specs=pl.BlockSpec((_ROWS_B, D), lambda i: (i, 0)),
        out_shape=jax.ShapeDtypeStruct((N, D), jnp.float32),
    )(x, deg2, w0)


_GPB = _ROWS_B // NPG
_ISQD = 1.0 / np.sqrt(D)


def _layer_tc_body(has_next, acc_ref, xsin_ref, deg_ref, bc_ref, wq_ref,
                   bq_ref, wk_ref, bk_ref, *rest):
    if has_next:
        wn_ref, ro_ref, xs_ref = rest
    else:
        (ro_ref,) = rest
    deg = deg_ref[...]
    dinv = lax.rsqrt(deg[0, :, 0:1] + deg[1, :, 0:1] + 1.0)
    acc = acc_ref[...]
    h = dinv * (acc[0] + acc[1] - xsin_ref[...]) + bc_ref[...]
    gm = jnp.mean(h.reshape(_GPB, NPG, D), axis=1)
    xq = jnp.dot(gm, wq_ref[...], preferred_element_type=jnp.float32) + bq_ref[...]
    xk = jnp.dot(h, wk_ref[...], preferred_element_type=jnp.float32) + bk_ref[...]
    xqe = jnp.broadcast_to(xq[:, None, :], (_GPB, NPG, D)).reshape(_ROWS_B, D)
    sc = jnp.sum(xk * xqe, axis=-1, keepdims=True) * _ISQD
    att = jax.nn.sigmoid(sc)
    ro_ref[...] = jnp.mean((h * att).reshape(_GPB, NPG, D), axis=1
                           ).reshape(1, _GPB, D)
    if has_next:
        xs_ref[...] = dinv * jnp.dot(h, wn_ref[...],
                                     preferred_element_type=jnp.float32)


def _layer_tc(acc, xsin, deg2, bc, wq, bq, wk, bk, wn):
    has_next = wn is not None
    full = lambda i: (0, 0)
    in_specs = [
        pl.BlockSpec((2, _ROWS_B, D), lambda i: (0, i, 0)),
        pl.BlockSpec((_ROWS_B, D), lambda i: (i, 0)),
        pl.BlockSpec((2, _ROWS_B, DW), lambda i: (0, i, 0)),
        pl.BlockSpec((1, D), full),
        pl.BlockSpec((D, D), full),
        pl.BlockSpec((1, D), full),
        pl.BlockSpec((D, D), full),
        pl.BlockSpec((1, D), full),
    ]
    out_specs = [pl.BlockSpec((1, _GPB, D), lambda i: (i, 0, 0))]
    out_shape = [jax.ShapeDtypeStruct((G // _GPB, _GPB, D), jnp.float32)]
    args = [acc, xsin, deg2, bc.reshape(1, D), wq, bq.reshape(1, D), wk,
            bk.reshape(1, D)]
    if has_next:
        in_specs.append(pl.BlockSpec((D, D), full))
        out_specs.append(pl.BlockSpec((_ROWS_B, D), lambda i: (i, 0)))
        out_shape.append(jax.ShapeDtypeStruct((N, D), jnp.float32))
        args.append(wn)
    return pl.pallas_call(
        functools.partial(_layer_tc_body, has_next),
        grid=(N // _ROWS_B,),
        in_specs=in_specs,
        out_specs=out_specs,
        out_shape=out_shape,
    )(*args)


def kernel(x, edge_index, batch, Wc0, bc0, Wq0, bq0, Wk0, bk0,
           Wc1, bc1, Wq1, bq1, Wk1, bk1, Wc2, bc2, Wq2, bq2, Wk2, bk2):
    src = edge_index[0]
    dst = edge_index[1]
    npd = EPAD - E
    pad_src = jnp.arange(npd, dtype=jnp.int32) % 64
    pad_dst = N + (jnp.arange(npd, dtype=jnp.int32) % (NPAD - N))
    srcr = jnp.concatenate([src, pad_src]).reshape(32, CHUNKS, CW)
    dstr = jnp.concatenate([dst, pad_dst]).reshape(32, CHUNKS, CW)
    ones = jnp.ones((CW, DW), jnp.float32)
    z1 = jnp.zeros((640, DW), jnp.float32)

    deg2 = _deg_sc(dstr, ones, z1)
    xs = _pre_tc(x, deg2, Wc0)
    params = [(bc0, Wq0, bq0, Wk0, bk0, Wc1),
              (bc1, Wq1, bq1, Wk1, bk1, Wc2),
              (bc2, Wq2, bq2, Wk2, bk2, None)]
    ros = []
    for bc, wq, bq, wk, bk, wn in params:
        acc = _scatter_sc(xs, srcr, dstr)
        res = _layer_tc(acc, xs, deg2, bc, wq, bq, wk, bk, wn)
        if wn is not None:
            ro, xs = res
        else:
            (ro,) = res
        ros.append(ro.reshape(G, D))
    return jnp.concatenate(ros, axis=1)


# speedup vs baseline: 1.1691x; 1.1691x over previous
import functools

import jax
import jax.numpy as jnp
import numpy as np
from jax import lax
from jax.experimental import pallas as pl
from jax.experimental.pallas import tpu as pltpu
from jax.experimental.pallas import tpu_sc as plsc

N = 10000
E = 320000
D = 128
G = 100
NPG = 100

NPAD = 10064
EPAD = 327680
CHUNKS = 80
CW = 128
IBLK = 16
DW = 16
DEGP = 10240

_MESH = plsc.VectorSubcoreMesh(core_axis_name="c", subcore_axis_name="s",
                               num_cores=2, num_subcores=16)


@functools.partial(
    pl.kernel,
    out_type=jax.ShapeDtypeStruct((2, DEGP, DW), jnp.float32),
    mesh=_MESH,
    compiler_params=pltpu.CompilerParams(use_tc_tiling_on_sc=False),
    scratch_types=[
        pltpu.MemorySpace.VMEM_SHARED((DEGP, DW), jnp.float32),
        pltpu.MemorySpace.VMEM((CHUNKS, CW), jnp.int32),
        pltpu.MemorySpace.VMEM((CW, DW), jnp.float32),
        pltpu.MemorySpace.VMEM((640, DW), jnp.float32),
    ],
)
def _deg_sc(dstr_hbm, ones_hbm, z1_hbm, out_hbm, dacc, dst_v, ones_v, z_v):
    c = lax.axis_index("c")
    s = lax.axis_index("s")
    wid = s * 2 + c
    pltpu.sync_copy(z1_hbm, z_v)
    pltpu.sync_copy(z_v, dacc.at[pl.ds(s * 640, 640)])
    pltpu.sync_copy(ones_hbm, ones_v)
    pltpu.sync_copy(dstr_hbm.at[wid], dst_v)
    plsc.subcore_barrier()

    def chunk(j, carry):
        pltpu.sync_copy(ones_v, dacc.at[dst_v.at[j]], add=True)
        return carry
    lax.fori_loop(0, CHUNKS, chunk, 0)
    plsc.subcore_barrier()
    pltpu.sync_copy(dacc.at[pl.ds(s * 640, 640)],
                    out_hbm.at[c, pl.ds(s * 640, 640)])


@functools.partial(
    pl.kernel,
    out_type=jax.ShapeDtypeStruct((2, N, D), jnp.float32),
    mesh=_MESH,
    compiler_params=pltpu.CompilerParams(use_tc_tiling_on_sc=False),
    scratch_types=[
        pltpu.MemorySpace.VMEM_SHARED((NPAD, D), jnp.float32),
        pltpu.MemorySpace.VMEM((IBLK, CW), jnp.int32),
        pltpu.MemorySpace.VMEM((IBLK, CW), jnp.int32),
        pltpu.MemorySpace.VMEM((CW, D), jnp.float32),
        pltpu.MemorySpace.VMEM((CW, D), jnp.float32),
        pltpu.SemaphoreType.DMA,
        pltpu.SemaphoreType.DMA,
    ],
)
def _scatter_sc(xs_hbm, srcr_hbm, dstr_hbm, out_hbm,
                acc_s, src_v, dst_v, rows0, rows1, g0, g1):
    c = lax.axis_index("c")
    s = lax.axis_index("s")
    wid = s * 2 + c
    def zrow(i, carry):
        for t in range(8):
            rows0[i, pl.ds(16 * t, 16)] = jnp.zeros((16,), jnp.float32)
        return carry
    lax.fori_loop(0, 80, zrow, 0)
    for j in range(8):
        i = s + 16 * j

        @pl.when(i < 125)
        def _():
            pltpu.sync_copy(rows0.at[pl.ds(0, 80)], acc_s.at[pl.ds(80 * i, 80)])
    plsc.subcore_barrier()

    def wait_g(rows, sem):
        pltpu.make_async_copy(xs_hbm.at[src_v.at[0]], rows, sem).wait()

    def oblk(k, carry):
        pltpu.sync_copy(srcr_hbm.at[wid, pl.ds(IBLK * k, IBLK)], src_v)
        pltpu.sync_copy(dstr_hbm.at[wid, pl.ds(IBLK * k, IBLK)], dst_v)
        pltpu.async_copy(xs_hbm.at[src_v.at[0]], rows0, g0)

        def pair(p, carry2):
            j = 2 * p
            pltpu.async_copy(xs_hbm.at[src_v.at[j + 1]], rows1, g1)
            wait_g(rows0, g0)
            pltpu.sync_copy(rows0, acc_s.at[dst_v.at[j]], add=True)

            @pl.when(j + 2 < IBLK)
            def _():
                pltpu.async_copy(xs_hbm.at[src_v.at[j + 2]], rows0, g0)
            wait_g(rows1, g1)
            pltpu.sync_copy(rows1, acc_s.at[dst_v.at[j + 1]], add=True)
            return carry2
        return lax.fori_loop(0, IBLK // 2, pair, carry)
    lax.fori_loop(0, CHUNKS // IBLK, oblk, 0)
    plsc.subcore_barrier()

    for j in range(8):
        i = s + 16 * j

        @pl.when(i < 125)
        def _():
            r0 = 80 * i
            pltpu.sync_copy(acc_s.at[pl.ds(r0, 80)], rows0.at[pl.ds(0, 80)])
            pltpu.sync_copy(rows0.at[pl.ds(0, 80)], out_hbm.at[c, pl.ds(r0, 80)])


_ROWS_B = 400


def _pre_tc_body(x_ref, deg_ref, w_ref, xs_ref):
    deg = deg_ref[...]
    dinv = lax.rsqrt(deg[0, :, 0:1] + deg[1, :, 0:1] + 1.0)
    xs_ref[...] = dinv * jnp.dot(x_ref[...], w_ref[...],
                                 preferred_element_type=jnp.float32)


def _pre_tc(x, deg2, w0):
    return pl.pallas_call(
        _pre_tc_body,
        grid=(N // _ROWS_B,),
        in_specs=[
            pl.BlockSpec((_ROWS_B, D), lambda i: (i, 0)),
            pl.BlockSpec((2, _ROWS_B, DW), lambda i: (0, i, 0)),
            pl.BlockSpec((D, D), lambda i: (0, 0)),
        ],
        out_specs=pl.BlockSpec((_ROWS_B, D), lambda i: (i, 0)),
        out_shape=jax.ShapeDtypeStruct((N, D), jnp.float32),
    )(x, deg2, w0)


_GPB = _ROWS_B // NPG
_ISQD = 1.0 / np.sqrt(D)


def _layer_tc_body(has_next, acc_ref, xsin_ref, deg_ref, bc_ref, wq_ref,
                   bq_ref, wk_ref, bk_ref, *rest):
    if has_next:
        wn_ref, ro_ref, xs_ref = rest
    else:
        (ro_ref,) = rest
    deg = deg_ref[...]
    dinv = lax.rsqrt(deg[0, :, 0:1] + deg[1, :, 0:1] + 1.0)
    acc = acc_ref[...]
    h = dinv * (acc[0] + acc[1] + xsin_ref[...]) + bc_ref[...]
    gm = jnp.mean(h.reshape(_GPB, NPG, D), axis=1)
    xq = jnp.dot(gm, wq_ref[...], preferred_element_type=jnp.float32) + bq_ref[...]
    xk = jnp.dot(h, wk_ref[...], preferred_element_type=jnp.float32) + bk_ref[...]
    xqe = jnp.broadcast_to(xq[:, None, :], (_GPB, NPG, D)).reshape(_ROWS_B, D)
    sc = jnp.sum(xk * xqe, axis=-1, keepdims=True) * _ISQD
    att = jax.nn.sigmoid(sc)
    ro_ref[...] = jnp.mean((h * att).reshape(_GPB, NPG, D), axis=1
                           ).reshape(1, _GPB, D)
    if has_next:
        xs_ref[...] = dinv * jnp.dot(h, wn_ref[...],
                                     preferred_element_type=jnp.float32)


def _layer_tc(acc, xsin, deg2, bc, wq, bq, wk, bk, wn):
    has_next = wn is not None
    full = lambda i: (0, 0)
    in_specs = [
        pl.BlockSpec((2, _ROWS_B, D), lambda i: (0, i, 0)),
        pl.BlockSpec((_ROWS_B, D), lambda i: (i, 0)),
        pl.BlockSpec((2, _ROWS_B, DW), lambda i: (0, i, 0)),
        pl.BlockSpec((1, D), full),
        pl.BlockSpec((D, D), full),
        pl.BlockSpec((1, D), full),
        pl.BlockSpec((D, D), full),
        pl.BlockSpec((1, D), full),
    ]
    out_specs = [pl.BlockSpec((1, _GPB, D), lambda i: (i, 0, 0))]
    out_shape = [jax.ShapeDtypeStruct((G // _GPB, _GPB, D), jnp.float32)]
    args = [acc, xsin, deg2, bc.reshape(1, D), wq, bq.reshape(1, D), wk,
            bk.reshape(1, D)]
    if has_next:
        in_specs.append(pl.BlockSpec((D, D), full))
        out_specs.append(pl.BlockSpec((_ROWS_B, D), lambda i: (i, 0)))
        out_shape.append(jax.ShapeDtypeStruct((N, D), jnp.float32))
        args.append(wn)
    return pl.pallas_call(
        functools.partial(_layer_tc_body, has_next),
        grid=(N // _ROWS_B,),
        in_specs=in_specs,
        out_specs=out_specs,
        out_shape=out_shape,
    )(*args)


def kernel(x, edge_index, batch, Wc0, bc0, Wq0, bq0, Wk0, bk0,
           Wc1, bc1, Wq1, bq1, Wk1, bk1, Wc2, bc2, Wq2, bq2, Wk2, bk2):
    src = edge_index[0]
    dst = edge_index[1]
    npd = EPAD - E
    pad_src = jnp.arange(npd, dtype=jnp.int32) % 64
    pad_dst = N + (jnp.arange(npd, dtype=jnp.int32) % (NPAD - N))
    srcr = jnp.concatenate([src, pad_src]).reshape(32, CHUNKS, CW)
    dstr = jnp.concatenate([dst, pad_dst]).reshape(32, CHUNKS, CW)
    ones = jnp.ones((CW, DW), jnp.float32)
    z1 = jnp.zeros((640, DW), jnp.float32)

    deg2 = _deg_sc(dstr, ones, z1)
    xs = _pre_tc(x, deg2, Wc0)
    params = [(bc0, Wq0, bq0, Wk0, bk0, Wc1),
              (bc1, Wq1, bq1, Wk1, bk1, Wc2),
              (bc2, Wq2, bq2, Wk2, bk2, None)]
    ros = []
    for bc, wq, bq, wk, bk, wn in params:
        acc = _scatter_sc(xs, srcr, dstr)
        res = _layer_tc(acc, xs, deg2, bc, wq, bq, wk, bk, wn)
        if wn is not None:
            ro, xs = res
        else:
            (ro,) = res
        ros.append(ro.reshape(G, D))
    return jnp.concatenate(ros, axis=1)


# speedup vs baseline: 1.2391x; 1.0598x over previous
import functools

import jax
import jax.numpy as jnp
import numpy as np
from jax import lax
from jax.experimental import pallas as pl
from jax.experimental.pallas import tpu as pltpu
from jax.experimental.pallas import tpu_sc as plsc

N = 10000
E = 320000
D = 128
G = 100
NPG = 100

NPAD = 10064
EPAD = 327680
CHUNKS = 80
CW = 128
IBLK = 40
DW = 16
DEGP = 10240

_MESH = plsc.VectorSubcoreMesh(core_axis_name="c", subcore_axis_name="s",
                               num_cores=2, num_subcores=16)


@functools.partial(
    pl.kernel,
    out_type=jax.ShapeDtypeStruct((2, DEGP, DW), jnp.float32),
    mesh=_MESH,
    compiler_params=pltpu.CompilerParams(use_tc_tiling_on_sc=False),
    scratch_types=[
        pltpu.MemorySpace.VMEM_SHARED((DEGP, DW), jnp.float32),
        pltpu.MemorySpace.VMEM((CHUNKS, CW), jnp.int32),
        pltpu.MemorySpace.VMEM((CW, DW), jnp.float32),
        pltpu.MemorySpace.VMEM((640, DW), jnp.float32),
        pltpu.SemaphoreType.DMA,
    ],
)
def _deg_sc(dstr_hbm, ones_hbm, z1_hbm, out_hbm, dacc, dst_v, ones_v, z_v, sd):
    c = lax.axis_index("c")
    s = lax.axis_index("s")
    wid = s * 2 + c
    pltpu.sync_copy(z1_hbm, z_v)
    pltpu.sync_copy(z_v, dacc.at[pl.ds(s * 640, 640)])
    pltpu.sync_copy(ones_hbm, ones_v)
    pltpu.sync_copy(dstr_hbm.at[wid], dst_v)
    plsc.subcore_barrier()

    def oblk(k, carry):
        for t in range(8):
            pltpu.async_copy(ones_v, dacc.at[dst_v.at[8 * k + t]], sd, add=True)
        for t in range(8):
            pltpu.make_async_copy(ones_v, dacc.at[dst_v.at[0]], sd).wait()
        return carry
    lax.fori_loop(0, CHUNKS // 8, oblk, 0)
    plsc.subcore_barrier()
    pltpu.sync_copy(dacc.at[pl.ds(s * 640, 640)],
                    out_hbm.at[c, pl.ds(s * 640, 640)])


@functools.partial(
    pl.kernel,
    out_type=jax.ShapeDtypeStruct((2, N, D), jnp.float32),
    mesh=_MESH,
    compiler_params=pltpu.CompilerParams(use_tc_tiling_on_sc=False),
    scratch_types=[
        pltpu.MemorySpace.VMEM_SHARED((NPAD, D), jnp.float32),
        pltpu.MemorySpace.VMEM((IBLK, CW), jnp.int32),
        pltpu.MemorySpace.VMEM((IBLK, CW), jnp.int32),
        pltpu.MemorySpace.VMEM((CW, D), jnp.float32),
        pltpu.MemorySpace.VMEM((CW, D), jnp.float32),
        pltpu.SemaphoreType.DMA,
        pltpu.SemaphoreType.DMA,
    ],
)
def _scatter_sc(xs_hbm, srcr_hbm, dstr_hbm, out_hbm,
                acc_s, src_v, dst_v, rows0, rows1, g0, g1):
    c = lax.axis_index("c")
    s = lax.axis_index("s")
    wid = s * 2 + c
    def zrow(i, carry):
        for t in range(8):
            rows0[i, pl.ds(16 * t, 16)] = jnp.zeros((16,), jnp.float32)
        return carry
    lax.fori_loop(0, 80, zrow, 0)
    for j in range(8):
        i = s + 16 * j

        @pl.when(i < 125)
        def _():
            pltpu.sync_copy(rows0.at[pl.ds(0, 80)], acc_s.at[pl.ds(80 * i, 80)])
    plsc.subcore_barrier()

    def wait_g(rows, sem):
        pltpu.make_async_copy(xs_hbm.at[src_v.at[0]], rows, sem).wait()

    def oblk(k, carry):
        pltpu.sync_copy(srcr_hbm.at[wid, pl.ds(IBLK * k, IBLK)], src_v)
        pltpu.sync_copy(dstr_hbm.at[wid, pl.ds(IBLK * k, IBLK)], dst_v)
        pltpu.async_copy(xs_hbm.at[src_v.at[0]], rows0, g0)

        def pair(p, carry2):
            j = 2 * p
            pltpu.async_copy(xs_hbm.at[src_v.at[j + 1]], rows1, g1)
            wait_g(rows0, g0)
            pltpu.sync_copy(rows0, acc_s.at[dst_v.at[j]], add=True)

            @pl.when(j + 2 < IBLK)
            def _():
                pltpu.async_copy(xs_hbm.at[src_v.at[j + 2]], rows0, g0)
            wait_g(rows1, g1)
            pltpu.sync_copy(rows1, acc_s.at[dst_v.at[j + 1]], add=True)
            return carry2
        return lax.fori_loop(0, IBLK // 2, pair, carry)
    lax.fori_loop(0, CHUNKS // IBLK, oblk, 0)
    plsc.subcore_barrier()

    for j in range(8):
        i = s + 16 * j

        @pl.when(i < 125)
        def _():
            r0 = 80 * i
            pltpu.sync_copy(acc_s.at[pl.ds(r0, 80)], rows0.at[pl.ds(0, 80)])
            pltpu.sync_copy(rows0.at[pl.ds(0, 80)], out_hbm.at[c, pl.ds(r0, 80)])


_ROWS_B = 400


def _pre_tc_body(x_ref, deg_ref, w_ref, xs_ref):
    deg = deg_ref[...]
    dinv = lax.rsqrt(deg[0, :, 0:1] + deg[1, :, 0:1] + 1.0)
    xs_ref[...] = dinv * jnp.dot(x_ref[...], w_ref[...],
                                 preferred_element_type=jnp.float32)


def _pre_tc(x, deg2, w0):
    return pl.pallas_call(
        _pre_tc_body,
        grid=(N // _ROWS_B,),
        in_specs=[
            pl.BlockSpec((_ROWS_B, D), lambda i: (i, 0)),
            pl.BlockSpec((2, _ROWS_B, DW), lambda i: (0, i, 0)),
            pl.BlockSpec((D, D), lambda i: (0, 0)),
        ],
        out_specs=pl.BlockSpec((_ROWS_B, D), lambda i: (i, 0)),
        out_shape=jax.ShapeDtypeStruct((N, D), jnp.float32),
    )(x, deg2, w0)


_GPB = _ROWS_B // NPG
_ISQD = 1.0 / np.sqrt(D)


def _layer_tc_body(has_next, acc_ref, xsin_ref, deg_ref, bc_ref, wq_ref,
                   bq_ref, wk_ref, bk_ref, *rest):
    if has_next:
        wn_ref, ro_ref, xs_ref = rest
    else:
        (ro_ref,) = rest
    deg = deg_ref[...]
    dinv = lax.rsqrt(deg[0, :, 0:1] + deg[1, :, 0:1] + 1.0)
    acc = acc_ref[...]
    h = dinv * (acc[0] + acc[1] + xsin_ref[...]) + bc_ref[...]
    gm = jnp.mean(h.reshape(_GPB, NPG, D), axis=1)
    xq = jnp.dot(gm, wq_ref[...], preferred_element_type=jnp.float32) + bq_ref[...]
    xk = jnp.dot(h, wk_ref[...], preferred_element_type=jnp.float32) + bk_ref[...]
    xqe = jnp.broadcast_to(xq[:, None, :], (_GPB, NPG, D)).reshape(_ROWS_B, D)
    sc = jnp.sum(xk * xqe, axis=-1, keepdims=True) * _ISQD
    att = jax.nn.sigmoid(sc)
    ro_ref[...] = jnp.mean((h * att).reshape(_GPB, NPG, D), axis=1
                           ).reshape(1, _GPB, D)
    if has_next:
        xs_ref[...] = dinv * jnp.dot(h, wn_ref[...],
                                     preferred_element_type=jnp.float32)


def _layer_tc(acc, xsin, deg2, bc, wq, bq, wk, bk, wn):
    has_next = wn is not None
    full = lambda i: (0, 0)
    in_specs = [
        pl.BlockSpec((2, _ROWS_B, D), lambda i: (0, i, 0)),
        pl.BlockSpec((_ROWS_B, D), lambda i: (i, 0)),
        pl.BlockSpec((2, _ROWS_B, DW), lambda i: (0, i, 0)),
        pl.BlockSpec((1, D), full),
        pl.BlockSpec((D, D), full),
        pl.BlockSpec((1, D), full),
        pl.BlockSpec((D, D), full),
        pl.BlockSpec((1, D), full),
    ]
    out_specs = [pl.BlockSpec((1, _GPB, D), lambda i: (i, 0, 0))]
    out_shape = [jax.ShapeDtypeStruct((G // _GPB, _GPB, D), jnp.float32)]
    args = [acc, xsin, deg2, bc.reshape(1, D), wq, bq.reshape(1, D), wk,
            bk.reshape(1, D)]
    if has_next:
        in_specs.append(pl.BlockSpec((D, D), full))
        out_specs.append(pl.BlockSpec((_ROWS_B, D), lambda i: (i, 0)))
        out_shape.append(jax.ShapeDtypeStruct((N, D), jnp.float32))
        args.append(wn)
    return pl.pallas_call(
        functools.partial(_layer_tc_body, has_next),
        grid=(N // _ROWS_B,),
        in_specs=in_specs,
        out_specs=out_specs,
        out_shape=out_shape,
    )(*args)


def kernel(x, edge_index, batch, Wc0, bc0, Wq0, bq0, Wk0, bk0,
           Wc1, bc1, Wq1, bq1, Wk1, bk1, Wc2, bc2, Wq2, bq2, Wk2, bk2):
    src = edge_index[0]
    dst = edge_index[1]
    npd = EPAD - E
    pad_src = jnp.arange(npd, dtype=jnp.int32) % 64
    pad_dst = N + (jnp.arange(npd, dtype=jnp.int32) % (NPAD - N))
    srcr = jnp.concatenate([src, pad_src]).reshape(32, CHUNKS, CW)
    dstr = jnp.concatenate([dst, pad_dst]).reshape(32, CHUNKS, CW)
    ones = jnp.ones((CW, DW), jnp.float32)
    z1 = jnp.zeros((640, DW), jnp.float32)

    deg2 = _deg_sc(dstr, ones, z1)
    xs = _pre_tc(x, deg2, Wc0)
    params = [(bc0, Wq0, bq0, Wk0, bk0, Wc1),
              (bc1, Wq1, bq1, Wk1, bk1, Wc2),
              (bc2, Wq2, bq2, Wk2, bk2, None)]
    ros = []
    for bc, wq, bq, wk, bk, wn in params:
        acc = _scatter_sc(xs, srcr, dstr)
        res = _layer_tc(acc, xs, deg2, bc, wq, bq, wk, bk, wn)
        if wn is not None:
            ro, xs = res
        else:
            (ro,) = res
        ros.append(ro.reshape(G, D))
    return jnp.concatenate(ros, axis=1)


# speedup vs baseline: 1.2613x; 1.0180x over previous
import functools

import jax
import jax.numpy as jnp
import numpy as np
from jax import lax
from jax.experimental import pallas as pl
from jax.experimental.pallas import tpu as pltpu
from jax.experimental.pallas import tpu_sc as plsc

N = 10000
E = 320000
D = 128
G = 100
NPG = 100

NPAD = 10064
EPAD = 327680
CHUNKS = 80
CW = 128
IBLK = 40
DW = 16
DEGP = 10240

_MESH = plsc.VectorSubcoreMesh(core_axis_name="c", subcore_axis_name="s",
                               num_cores=2, num_subcores=16)


@functools.partial(
    pl.kernel,
    out_type=jax.ShapeDtypeStruct((2, DEGP, DW), jnp.float32),
    mesh=_MESH,
    compiler_params=pltpu.CompilerParams(use_tc_tiling_on_sc=False),
    scratch_types=[
        pltpu.MemorySpace.VMEM_SHARED((DEGP, DW), jnp.float32),
        pltpu.MemorySpace.VMEM((CHUNKS, CW), jnp.int32),
        pltpu.MemorySpace.VMEM((CW, DW), jnp.float32),
        pltpu.MemorySpace.VMEM((640, DW), jnp.float32),
        pltpu.SemaphoreType.DMA,
    ],
)
def _deg_sc(dstr_hbm, ones_hbm, z1_hbm, out_hbm, dacc, dst_v, ones_v, z_v, sd):
    c = lax.axis_index("c")
    s = lax.axis_index("s")
    wid = s * 2 + c
    pltpu.sync_copy(z1_hbm, z_v)
    pltpu.sync_copy(z_v, dacc.at[pl.ds(s * 640, 640)])
    pltpu.sync_copy(ones_hbm, ones_v)
    pltpu.sync_copy(dstr_hbm.at[wid], dst_v)
    plsc.subcore_barrier()

    def oblk(k, carry):
        for t in range(8):
            pltpu.async_copy(ones_v, dacc.at[dst_v.at[8 * k + t]], sd, add=True)
        for t in range(8):
            pltpu.make_async_copy(ones_v, dacc.at[dst_v.at[0]], sd).wait()
        return carry
    lax.fori_loop(0, CHUNKS // 8, oblk, 0)
    plsc.subcore_barrier()
    pltpu.sync_copy(dacc.at[pl.ds(s * 640, 640)],
                    out_hbm.at[c, pl.ds(s * 640, 640)])


@functools.partial(
    pl.kernel,
    out_type=jax.ShapeDtypeStruct((2, N, D), jnp.float32),
    mesh=_MESH,
    compiler_params=pltpu.CompilerParams(use_tc_tiling_on_sc=False),
    scratch_types=[
        pltpu.MemorySpace.VMEM_SHARED((NPAD, D), jnp.float32),
        pltpu.MemorySpace.VMEM((IBLK, CW), jnp.int32),
        pltpu.MemorySpace.VMEM((IBLK, CW), jnp.int32),
        pltpu.MemorySpace.VMEM((CW, D), jnp.float32),
        pltpu.MemorySpace.VMEM((CW, D), jnp.float32),
        pltpu.SemaphoreType.DMA,
        pltpu.SemaphoreType.DMA,
        pltpu.SemaphoreType.DMA,
        pltpu.SemaphoreType.DMA,
    ],
)
def _scatter_sc(xs_hbm, srcr_hbm, dstr_hbm, out_hbm,
                acc_s, src_v, dst_v, rows0, rows1, g0, g1, s0, s1):
    c = lax.axis_index("c")
    s = lax.axis_index("s")
    wid = s * 2 + c
    def zrow(i, carry):
        for t in range(8):
            rows0[i, pl.ds(16 * t, 16)] = jnp.zeros((16,), jnp.float32)
        return carry
    lax.fori_loop(0, 80, zrow, 0)
    for j in range(8):
        i = s + 16 * j

        @pl.when(i < 125)
        def _():
            pltpu.async_copy(rows0.at[pl.ds(0, 80)], acc_s.at[pl.ds(80 * i, 80)], g0)
    for j in range(8):
        i = s + 16 * j

        @pl.when(i < 125)
        def _():
            pltpu.make_async_copy(rows0.at[pl.ds(0, 80)],
                                  acc_s.at[pl.ds(0, 80)], g0).wait()
    plsc.subcore_barrier()

    def wait_g(rows, sem):
        pltpu.make_async_copy(xs_hbm.at[src_v.at[0]], rows, sem).wait()

    def oblk(k, carry):
        pltpu.sync_copy(srcr_hbm.at[wid, pl.ds(IBLK * k, IBLK)], src_v)
        pltpu.sync_copy(dstr_hbm.at[wid, pl.ds(IBLK * k, IBLK)], dst_v)
        pltpu.async_copy(xs_hbm.at[src_v.at[0]], rows0, g0)

        def pair(p, carry2):
            j = 2 * p
            pltpu.async_copy(xs_hbm.at[src_v.at[j + 1]], rows1, g1)
            wait_g(rows0, g0)
            pltpu.sync_copy(rows0, acc_s.at[dst_v.at[j]], add=True)

            @pl.when(j + 2 < IBLK)
            def _():
                pltpu.async_copy(xs_hbm.at[src_v.at[j + 2]], rows0, g0)
            wait_g(rows1, g1)
            pltpu.sync_copy(rows1, acc_s.at[dst_v.at[j + 1]], add=True)
            return carry2
        return lax.fori_loop(0, IBLK // 2, pair, carry)
    lax.fori_loop(0, CHUNKS // IBLK, oblk, 0)
    plsc.subcore_barrier()

    bufs, gsems, wsems = (rows0, rows1), (g0, g1), (s0, s1)
    for j in range(8):
        i = s + 16 * j
        rv, gs, ws = bufs[j % 2], gsems[j % 2], wsems[j % 2]

        @pl.when(i < 125)
        def _():
            r0 = 80 * i
            if j >= 2:
                pltpu.make_async_copy(rv.at[pl.ds(0, 80)],
                                      out_hbm.at[c, pl.ds(0, 80)], ws).wait()
            pltpu.async_copy(acc_s.at[pl.ds(r0, 80)], rv.at[pl.ds(0, 80)], gs)
            pltpu.make_async_copy(acc_s.at[pl.ds(0, 80)],
                                  rv.at[pl.ds(0, 80)], gs).wait()
            pltpu.async_copy(rv.at[pl.ds(0, 80)], out_hbm.at[c, pl.ds(r0, 80)], ws)
    for b in range(2):
        pltpu.make_async_copy(bufs[b].at[pl.ds(0, 80)],
                              out_hbm.at[c, pl.ds(0, 80)], wsems[b]).wait()


_ROWS_B = 400


def _pre_tc_body(x_ref, deg_ref, w_ref, xs_ref):
    deg = deg_ref[...]
    dinv = lax.rsqrt(deg[0, :, 0:1] + deg[1, :, 0:1] + 1.0)
    xs_ref[...] = dinv * jnp.dot(x_ref[...], w_ref[...],
                                 preferred_element_type=jnp.float32)


def _pre_tc(x, deg2, w0):
    return pl.pallas_call(
        _pre_tc_body,
        grid=(N // _ROWS_B,),
        in_specs=[
            pl.BlockSpec((_ROWS_B, D), lambda i: (i, 0)),
            pl.BlockSpec((2, _ROWS_B, DW), lambda i: (0, i, 0)),
            pl.BlockSpec((D, D), lambda i: (0, 0)),
        ],
        out_specs=pl.BlockSpec((_ROWS_B, D), lambda i: (i, 0)),
        out_shape=jax.ShapeDtypeStruct((N, D), jnp.float32),
    )(x, deg2, w0)


_GPB = _ROWS_B // NPG
_ISQD = 1.0 / np.sqrt(D)


def _layer_tc_body(has_next, acc_ref, xsin_ref, deg_ref, bc_ref, wq_ref,
                   bq_ref, wk_ref, bk_ref, *rest):
    if has_next:
        wn_ref, ro_ref, xs_ref = rest
    else:
        (ro_ref,) = rest
    deg = deg_ref[...]
    dinv = lax.rsqrt(deg[0, :, 0:1] + deg[1, :, 0:1] + 1.0)
    acc = acc_ref[...]
    h = dinv * (acc[0] + acc[1] + xsin_ref[...]) + bc_ref[...]
    gm = jnp.mean(h.reshape(_GPB, NPG, D), axis=1)
    xq = jnp.dot(gm, wq_ref[...], preferred_element_type=jnp.float32) + bq_ref[...]
    xk = jnp.dot(h, wk_ref[...], preferred_element_type=jnp.float32) + bk_ref[...]
    xqe = jnp.broadcast_to(xq[:, None, :], (_GPB, NPG, D)).reshape(_ROWS_B, D)
    sc = jnp.sum(xk * xqe, axis=-1, keepdims=True) * _ISQD
    att = jax.nn.sigmoid(sc)
    ro_ref[...] = jnp.mean((h * att).reshape(_GPB, NPG, D), axis=1
                           ).reshape(1, _GPB, D)
    if has_next:
        xs_ref[...] = dinv * jnp.dot(h, wn_ref[...],
                                     preferred_element_type=jnp.float32)


def _layer_tc(acc, xsin, deg2, bc, wq, bq, wk, bk, wn):
    has_next = wn is not None
    full = lambda i: (0, 0)
    in_specs = [
        pl.BlockSpec((2, _ROWS_B, D), lambda i: (0, i, 0)),
        pl.BlockSpec((_ROWS_B, D), lambda i: (i, 0)),
        pl.BlockSpec((2, _ROWS_B, DW), lambda i: (0, i, 0)),
        pl.BlockSpec((1, D), full),
        pl.BlockSpec((D, D), full),
        pl.BlockSpec((1, D), full),
        pl.BlockSpec((D, D), full),
        pl.BlockSpec((1, D), full),
    ]
    out_specs = [pl.BlockSpec((1, _GPB, D), lambda i: (i, 0, 0))]
    out_shape = [jax.ShapeDtypeStruct((G // _GPB, _GPB, D), jnp.float32)]
    args = [acc, xsin, deg2, bc.reshape(1, D), wq, bq.reshape(1, D), wk,
            bk.reshape(1, D)]
    if has_next:
        in_specs.append(pl.BlockSpec((D, D), full))
        out_specs.append(pl.BlockSpec((_ROWS_B, D), lambda i: (i, 0)))
        out_shape.append(jax.ShapeDtypeStruct((N, D), jnp.float32))
        args.append(wn)
    return pl.pallas_call(
        functools.partial(_layer_tc_body, has_next),
        grid=(N // _ROWS_B,),
        in_specs=in_specs,
        out_specs=out_specs,
        out_shape=out_shape,
    )(*args)


def kernel(x, edge_index, batch, Wc0, bc0, Wq0, bq0, Wk0, bk0,
           Wc1, bc1, Wq1, bq1, Wk1, bk1, Wc2, bc2, Wq2, bq2, Wk2, bk2):
    src = edge_index[0]
    dst = edge_index[1]
    npd = EPAD - E
    pad_src = jnp.arange(npd, dtype=jnp.int32) % 64
    pad_dst = N + (jnp.arange(npd, dtype=jnp.int32) % (NPAD - N))
    srcr = jnp.concatenate([src, pad_src]).reshape(32, CHUNKS, CW)
    dstr = jnp.concatenate([dst, pad_dst]).reshape(32, CHUNKS, CW)
    ones = jnp.ones((CW, DW), jnp.float32)
    z1 = jnp.zeros((640, DW), jnp.float32)

    deg2 = _deg_sc(dstr, ones, z1)
    xs = _pre_tc(x, deg2, Wc0)
    params = [(bc0, Wq0, bq0, Wk0, bk0, Wc1),
              (bc1, Wq1, bq1, Wk1, bk1, Wc2),
              (bc2, Wq2, bq2, Wk2, bk2, None)]
    ros = []
    for bc, wq, bq, wk, bk, wn in params:
        acc = _scatter_sc(xs, srcr, dstr)
        res = _layer_tc(acc, xs, deg2, bc, wq, bq, wk, bk, wn)
        if wn is not None:
            ro, xs = res
        else:
            (ro,) = res
        ros.append(ro.reshape(G, D))
    return jnp.concatenate(ros, axis=1)


# speedup vs baseline: 1.3601x; 1.0783x over previous
import functools

import jax
import jax.numpy as jnp
import numpy as np
from jax import lax
from jax.experimental import pallas as pl
from jax.experimental.pallas import tpu as pltpu
from jax.experimental.pallas import tpu_sc as plsc

N = 10000
E = 320000
D = 128
G = 100
NPG = 100

NPAD = 10064
EPAD = 327680
CHUNKS = 80
CW = 128
IBLK = 40
DW = 16
DEGP = 10240

_MESH = plsc.VectorSubcoreMesh(core_axis_name="c", subcore_axis_name="s",
                               num_cores=2, num_subcores=16)


@functools.partial(
    pl.kernel,
    out_type=jax.ShapeDtypeStruct((2, DEGP, DW), jnp.float32),
    mesh=_MESH,
    compiler_params=pltpu.CompilerParams(use_tc_tiling_on_sc=False),
    scratch_types=[
        pltpu.MemorySpace.VMEM_SHARED((DEGP, DW), jnp.float32),
        pltpu.MemorySpace.VMEM((CHUNKS, CW), jnp.int32),
        pltpu.MemorySpace.VMEM((CW, DW), jnp.float32),
        pltpu.MemorySpace.VMEM((640, DW), jnp.float32),
        pltpu.SemaphoreType.DMA,
    ],
)
def _deg_sc(dstr_hbm, ones_hbm, z1_hbm, out_hbm, dacc, dst_v, ones_v, z_v, sd):
    c = lax.axis_index("c")
    s = lax.axis_index("s")
    wid = s * 2 + c
    pltpu.sync_copy(z1_hbm, z_v)
    pltpu.sync_copy(z_v, dacc.at[pl.ds(s * 640, 640)])
    pltpu.sync_copy(ones_hbm, ones_v)
    pltpu.sync_copy(dstr_hbm.at[wid], dst_v)
    plsc.subcore_barrier()

    def oblk(k, carry):
        for t in range(8):
            pltpu.async_copy(ones_v, dacc.at[dst_v.at[8 * k + t]], sd, add=True)
        for t in range(8):
            pltpu.make_async_copy(ones_v, dacc.at[dst_v.at[0]], sd).wait()
        return carry
    lax.fori_loop(0, CHUNKS // 8, oblk, 0)
    plsc.subcore_barrier()
    pltpu.sync_copy(dacc.at[pl.ds(s * 640, 640)],
                    out_hbm.at[c, pl.ds(s * 640, 640)])


@functools.partial(
    pl.kernel,
    out_type=jax.ShapeDtypeStruct((2, N, D), jnp.float32),
    mesh=_MESH,
    compiler_params=pltpu.CompilerParams(use_tc_tiling_on_sc=False),
    scratch_types=[
        pltpu.MemorySpace.VMEM_SHARED((NPAD, D), jnp.float32),
        pltpu.MemorySpace.VMEM((IBLK, CW), jnp.int32),
        pltpu.MemorySpace.VMEM((IBLK, CW), jnp.int32),
        pltpu.MemorySpace.VMEM((CW, D), jnp.float32),
        pltpu.MemorySpace.VMEM((CW, D), jnp.float32),
        pltpu.SemaphoreType.DMA,
        pltpu.SemaphoreType.DMA,
        pltpu.SemaphoreType.DMA,
        pltpu.SemaphoreType.DMA,
    ],
)
def _scatter_sc(xs_hbm, srcr_hbm, dstr_hbm, out_hbm,
                acc_s, src_v, dst_v, rows0, rows1, g0, g1, s0, s1):
    c = lax.axis_index("c")
    s = lax.axis_index("s")
    wid = s * 2 + c
    def zrow(i, carry):
        for t in range(8):
            rows0[i, pl.ds(16 * t, 16)] = jnp.zeros((16,), jnp.float32)
        return carry
    lax.fori_loop(0, 80, zrow, 0)
    for j in range(8):
        i = s + 16 * j

        @pl.when(i < 125)
        def _():
            pltpu.async_copy(rows0.at[pl.ds(0, 80)], acc_s.at[pl.ds(80 * i, 80)], g0)
    for j in range(8):
        i = s + 16 * j

        @pl.when(i < 125)
        def _():
            pltpu.make_async_copy(rows0.at[pl.ds(0, 80)],
                                  acc_s.at[pl.ds(0, 80)], g0).wait()
    plsc.subcore_barrier()

    def wait_g(rows, sem):
        pltpu.make_async_copy(xs_hbm.at[src_v.at[0]], rows, sem).wait()

    def oblk(k, carry):
        pltpu.sync_copy(srcr_hbm.at[wid, pl.ds(IBLK * k, IBLK)], src_v)
        pltpu.sync_copy(dstr_hbm.at[wid, pl.ds(IBLK * k, IBLK)], dst_v)
        pltpu.async_copy(xs_hbm.at[src_v.at[0]], rows0, g0)

        def pair(p, carry2):
            j = 2 * p
            pltpu.async_copy(xs_hbm.at[src_v.at[j + 1]], rows1, g1)
            wait_g(rows0, g0)
            pltpu.sync_copy(rows0, acc_s.at[dst_v.at[j]], add=True)

            @pl.when(j + 2 < IBLK)
            def _():
                pltpu.async_copy(xs_hbm.at[src_v.at[j + 2]], rows0, g0)
            wait_g(rows1, g1)
            pltpu.sync_copy(rows1, acc_s.at[dst_v.at[j + 1]], add=True)
            return carry2
        return lax.fori_loop(0, IBLK // 2, pair, carry)
    lax.fori_loop(0, CHUNKS // IBLK, oblk, 0)
    plsc.subcore_barrier()

    bufs, gsems, wsems = (rows0, rows1), (g0, g1), (s0, s1)
    for j in range(8):
        i = s + 16 * j
        rv, gs, ws = bufs[j % 2], gsems[j % 2], wsems[j % 2]

        @pl.when(i < 125)
        def _():
            r0 = 80 * i
            if j >= 2:
                pltpu.make_async_copy(rv.at[pl.ds(0, 80)],
                                      out_hbm.at[c, pl.ds(0, 80)], ws).wait()
            pltpu.async_copy(acc_s.at[pl.ds(r0, 80)], rv.at[pl.ds(0, 80)], gs)
            pltpu.make_async_copy(acc_s.at[pl.ds(0, 80)],
                                  rv.at[pl.ds(0, 80)], gs).wait()
            pltpu.async_copy(rv.at[pl.ds(0, 80)], out_hbm.at[c, pl.ds(r0, 80)], ws)
    for b in range(2):
        pltpu.make_async_copy(bufs[b].at[pl.ds(0, 80)],
                              out_hbm.at[c, pl.ds(0, 80)], wsems[b]).wait()


_ROWS_B = 1000


def _pre_tc_body(x_ref, deg_ref, w_ref, xs_ref):
    deg = deg_ref[...]
    dinv = lax.rsqrt(deg[0, :, 0:1] + deg[1, :, 0:1] + 1.0)
    xs_ref[...] = dinv * jnp.dot(x_ref[...], w_ref[...],
                                 preferred_element_type=jnp.float32)


def _pre_tc(x, deg2, w0):
    return pl.pallas_call(
        _pre_tc_body,
        grid=(N // _ROWS_B,),
        in_specs=[
            pl.BlockSpec((_ROWS_B, D), lambda i: (i, 0)),
            pl.BlockSpec((2, _ROWS_B, DW), lambda i: (0, i, 0)),
            pl.BlockSpec((D, D), lambda i: (0, 0)),
        ],
        out_specs=pl.BlockSpec((_ROWS_B, D), lambda i: (i, 0)),
        out_shape=jax.ShapeDtypeStruct((N, D), jnp.float32),
    )(x, deg2, w0)


_GPB = _ROWS_B // NPG
_ISQD = 1.0 / np.sqrt(D)


def _layer_tc_body(has_next, acc_ref, xsin_ref, deg_ref, bc_ref, wq_ref,
                   bq_ref, wk_ref, bk_ref, *rest):
    if has_next:
        wn_ref, ro_ref, xs_ref = rest
    else:
        (ro_ref,) = rest
    deg = deg_ref[...]
    dinv = lax.rsqrt(deg[0, :, 0:1] + deg[1, :, 0:1] + 1.0)
    acc = acc_ref[...]
    h = dinv * (acc[0] + acc[1] + xsin_ref[...]) + bc_ref[...]
    gm = jnp.mean(h.reshape(_GPB, NPG, D), axis=1)
    xq = jnp.dot(gm, wq_ref[...], preferred_element_type=jnp.float32) + bq_ref[...]
    xk = jnp.dot(h, wk_ref[...], preferred_element_type=jnp.float32) + bk_ref[...]
    xqe = jnp.broadcast_to(xq[:, None, :], (_GPB, NPG, D)).reshape(_ROWS_B, D)
    sc = jnp.sum(xk * xqe, axis=-1, keepdims=True) * _ISQD
    att = jax.nn.sigmoid(sc)
    ro_ref[...] = jnp.mean((h * att).reshape(_GPB, NPG, D), axis=1
                           ).reshape(1, _GPB, D)
    if has_next:
        xs_ref[...] = dinv * jnp.dot(h, wn_ref[...],
                                     preferred_element_type=jnp.float32)


def _layer_tc(acc, xsin, deg2, bc, wq, bq, wk, bk, wn):
    has_next = wn is not None
    full = lambda i: (0, 0)
    in_specs = [
        pl.BlockSpec((2, _ROWS_B, D), lambda i: (0, i, 0)),
        pl.BlockSpec((_ROWS_B, D), lambda i: (i, 0)),
        pl.BlockSpec((2, _ROWS_B, DW), lambda i: (0, i, 0)),
        pl.BlockSpec((1, D), full),
        pl.BlockSpec((D, D), full),
        pl.BlockSpec((1, D), full),
        pl.BlockSpec((D, D), full),
        pl.BlockSpec((1, D), full),
    ]
    out_specs = [pl.BlockSpec((1, _GPB, D), lambda i: (i, 0, 0))]
    out_shape = [jax.ShapeDtypeStruct((G // _GPB, _GPB, D), jnp.float32)]
    args = [acc, xsin, deg2, bc.reshape(1, D), wq, bq.reshape(1, D), wk,
            bk.reshape(1, D)]
    if has_next:
        in_specs.append(pl.BlockSpec((D, D), full))
        out_specs.append(pl.BlockSpec((_ROWS_B, D), lambda i: (i, 0)))
        out_shape.append(jax.ShapeDtypeStruct((N, D), jnp.float32))
        args.append(wn)
    return pl.pallas_call(
        functools.partial(_layer_tc_body, has_next),
        grid=(N // _ROWS_B,),
        in_specs=in_specs,
        out_specs=out_specs,
        out_shape=out_shape,
    )(*args)


def kernel(x, edge_index, batch, Wc0, bc0, Wq0, bq0, Wk0, bk0,
           Wc1, bc1, Wq1, bq1, Wk1, bk1, Wc2, bc2, Wq2, bq2, Wk2, bk2):
    src = edge_index[0]
    dst = edge_index[1]
    npd = EPAD - E
    pad_src = jnp.arange(npd, dtype=jnp.int32) % 64
    pad_dst = N + (jnp.arange(npd, dtype=jnp.int32) % (NPAD - N))
    srcr = jnp.concatenate([src, pad_src]).reshape(32, CHUNKS, CW)
    dstr = jnp.concatenate([dst, pad_dst]).reshape(32, CHUNKS, CW)
    ones = jnp.ones((CW, DW), jnp.float32)
    z1 = jnp.zeros((640, DW), jnp.float32)

    deg2 = _deg_sc(dstr, ones, z1)
    xs = _pre_tc(x, deg2, Wc0)
    params = [(bc0, Wq0, bq0, Wk0, bk0, Wc1),
              (bc1, Wq1, bq1, Wk1, bk1, Wc2),
              (bc2, Wq2, bq2, Wk2, bk2, None)]
    ros = []
    for bc, wq, bq, wk, bk, wn in params:
        acc = _scatter_sc(xs, srcr, dstr)
        res = _layer_tc(acc, xs, deg2, bc, wq, bq, wk, bk, wn)
        if wn is not None:
            ro, xs = res
        else:
            (ro,) = res
        ros.append(ro.reshape(G, D))
    return jnp.concatenate(ros, axis=1)


# speedup vs baseline: 1.3859x; 1.0189x over previous
import functools

import jax
import jax.numpy as jnp
import numpy as np
from jax import lax
from jax.experimental import pallas as pl
from jax.experimental.pallas import tpu as pltpu
from jax.experimental.pallas import tpu_sc as plsc

N = 10000
E = 320000
D = 128
G = 100
NPG = 100

NPAD = 10064
EPAD = 327680
CHUNKS = 80
CW = 128
IBLK = 40
DW = 16
DEGP = 10240

_MESH = plsc.VectorSubcoreMesh(core_axis_name="c", subcore_axis_name="s",
                               num_cores=2, num_subcores=16)


@functools.partial(
    pl.kernel,
    out_type=jax.ShapeDtypeStruct((2, DEGP, DW), jnp.float32),
    mesh=_MESH,
    compiler_params=pltpu.CompilerParams(use_tc_tiling_on_sc=False),
    scratch_types=[
        pltpu.MemorySpace.VMEM_SHARED((DEGP, DW), jnp.float32),
        pltpu.MemorySpace.VMEM((CHUNKS, CW), jnp.int32),
        pltpu.MemorySpace.VMEM((CW, DW), jnp.float32),
        pltpu.MemorySpace.VMEM((640, DW), jnp.float32),
        pltpu.SemaphoreType.DMA,
    ],
)
def _deg_sc(eir_hbm, ones_hbm, z1_hbm, out_hbm, dacc, dst_v, ones_v, z_v, sd):
    c = lax.axis_index("c")
    s = lax.axis_index("s")
    wid = s * 2 + c
    pltpu.sync_copy(z1_hbm, z_v)
    pltpu.sync_copy(z_v, dacc.at[pl.ds(s * 640, 640)])
    pltpu.sync_copy(ones_hbm, ones_v)
    pltpu.sync_copy(eir_hbm.at[1, wid], dst_v)
    plsc.subcore_barrier()

    def oblk(k, carry):
        for t in range(8):
            pltpu.async_copy(ones_v, dacc.at[dst_v.at[8 * k + t]], sd, add=True)
        for t in range(8):
            pltpu.make_async_copy(ones_v, dacc.at[dst_v.at[0]], sd).wait()
        return carry
    lax.fori_loop(0, CHUNKS // 8, oblk, 0)
    plsc.subcore_barrier()
    pltpu.sync_copy(dacc.at[pl.ds(s * 640, 640)],
                    out_hbm.at[c, pl.ds(s * 640, 640)])


@functools.partial(
    pl.kernel,
    out_type=jax.ShapeDtypeStruct((2, N, D), jnp.float32),
    mesh=_MESH,
    compiler_params=pltpu.CompilerParams(use_tc_tiling_on_sc=False),
    scratch_types=[
        pltpu.MemorySpace.VMEM_SHARED((NPAD, D), jnp.float32),
        pltpu.MemorySpace.VMEM((IBLK, CW), jnp.int32),
        pltpu.MemorySpace.VMEM((IBLK, CW), jnp.int32),
        pltpu.MemorySpace.VMEM((CW, D), jnp.float32),
        pltpu.MemorySpace.VMEM((CW, D), jnp.float32),
        pltpu.SemaphoreType.DMA,
        pltpu.SemaphoreType.DMA,
        pltpu.SemaphoreType.DMA,
        pltpu.SemaphoreType.DMA,
    ],
)
def _scatter_sc(xs_hbm, eir_hbm, out_hbm,
                acc_s, src_v, dst_v, rows0, rows1, g0, g1, s0, s1):
    c = lax.axis_index("c")
    s = lax.axis_index("s")
    wid = s * 2 + c
    def zrow(i, carry):
        for t in range(8):
            rows0[i, pl.ds(16 * t, 16)] = jnp.zeros((16,), jnp.float32)
        return carry
    lax.fori_loop(0, 80, zrow, 0)
    for j in range(8):
        i = s + 16 * j

        @pl.when(i < 125)
        def _():
            pltpu.async_copy(rows0.at[pl.ds(0, 80)], acc_s.at[pl.ds(80 * i, 80)], g0)
    for j in range(8):
        i = s + 16 * j

        @pl.when(i < 125)
        def _():
            pltpu.make_async_copy(rows0.at[pl.ds(0, 80)],
                                  acc_s.at[pl.ds(0, 80)], g0).wait()
    plsc.subcore_barrier()

    def wait_g(rows, sem):
        pltpu.make_async_copy(xs_hbm.at[src_v.at[0]], rows, sem).wait()

    def oblk(k, carry):
        pltpu.sync_copy(eir_hbm.at[0, wid, pl.ds(IBLK * k, IBLK)], src_v)
        pltpu.sync_copy(eir_hbm.at[1, wid, pl.ds(IBLK * k, IBLK)], dst_v)
        pltpu.async_copy(xs_hbm.at[src_v.at[0]], rows0, g0)

        def pair(p, carry2):
            j = 2 * p
            pltpu.async_copy(xs_hbm.at[src_v.at[j + 1]], rows1, g1)
            wait_g(rows0, g0)
            pltpu.sync_copy(rows0, acc_s.at[dst_v.at[j]], add=True)

            @pl.when(j + 2 < IBLK)
            def _():
                pltpu.async_copy(xs_hbm.at[src_v.at[j + 2]], rows0, g0)
            wait_g(rows1, g1)
            pltpu.sync_copy(rows1, acc_s.at[dst_v.at[j + 1]], add=True)
            return carry2
        return lax.fori_loop(0, IBLK // 2, pair, carry)
    lax.fori_loop(0, CHUNKS // IBLK, oblk, 0)
    plsc.subcore_barrier()

    bufs, gsems, wsems = (rows0, rows1), (g0, g1), (s0, s1)
    for j in range(8):
        i = s + 16 * j
        rv, gs, ws = bufs[j % 2], gsems[j % 2], wsems[j % 2]

        @pl.when(i < 125)
        def _():
            r0 = 80 * i
            if j >= 2:
                pltpu.make_async_copy(rv.at[pl.ds(0, 80)],
                                      out_hbm.at[c, pl.ds(0, 80)], ws).wait()
            pltpu.async_copy(acc_s.at[pl.ds(r0, 80)], rv.at[pl.ds(0, 80)], gs)
            pltpu.make_async_copy(acc_s.at[pl.ds(0, 80)],
                                  rv.at[pl.ds(0, 80)], gs).wait()
            pltpu.async_copy(rv.at[pl.ds(0, 80)], out_hbm.at[c, pl.ds(r0, 80)], ws)
    for b in range(2):
        pltpu.make_async_copy(bufs[b].at[pl.ds(0, 80)],
                              out_hbm.at[c, pl.ds(0, 80)], wsems[b]).wait()


_ROWS_B = 1000


def _pre_tc_body(x_ref, deg_ref, w_ref, xs_ref):
    deg = deg_ref[...]
    dinv = lax.rsqrt(deg[0, :, 0:1] + deg[1, :, 0:1] + 1.0)
    xs_ref[...] = dinv * jnp.dot(x_ref[...], w_ref[...],
                                 preferred_element_type=jnp.float32)


def _pre_tc(x, deg2, w0):
    return pl.pallas_call(
        _pre_tc_body,
        grid=(N // _ROWS_B,),
        in_specs=[
            pl.BlockSpec((_ROWS_B, D), lambda i: (i, 0)),
            pl.BlockSpec((2, _ROWS_B, DW), lambda i: (0, i, 0)),
            pl.BlockSpec((D, D), lambda i: (0, 0)),
        ],
        out_specs=pl.BlockSpec((_ROWS_B, D), lambda i: (i, 0)),
        out_shape=jax.ShapeDtypeStruct((N, D), jnp.float32),
    )(x, deg2, w0)


_GPB = _ROWS_B // NPG
_ISQD = 1.0 / np.sqrt(D)


def _layer_tc_body(has_next, acc_ref, xsin_ref, deg_ref, bc_ref, wq_ref,
                   bq_ref, wk_ref, bk_ref, *rest):
    if has_next:
        wn_ref, ro_ref, xs_ref = rest
    else:
        (ro_ref,) = rest
    deg = deg_ref[...]
    dinv = lax.rsqrt(deg[0, :, 0:1] + deg[1, :, 0:1] + 1.0)
    acc = acc_ref[...]
    h = dinv * (acc[0] + acc[1] + xsin_ref[...]) + bc_ref[...]
    gm = jnp.mean(h.reshape(_GPB, NPG, D), axis=1)
    xq = jnp.dot(gm, wq_ref[...], preferred_element_type=jnp.float32) + bq_ref[...]
    xk = jnp.dot(h, wk_ref[...], preferred_element_type=jnp.float32) + bk_ref[...]
    xqe = jnp.broadcast_to(xq[:, None, :], (_GPB, NPG, D)).reshape(_ROWS_B, D)
    sc = jnp.sum(xk * xqe, axis=-1, keepdims=True) * _ISQD
    att = jax.nn.sigmoid(sc)
    ro_ref[...] = jnp.mean((h * att).reshape(_GPB, NPG, D), axis=1
                           ).reshape(1, _GPB, D)
    if has_next:
        xs_ref[...] = dinv * jnp.dot(h, wn_ref[...],
                                     preferred_element_type=jnp.float32)


def _layer_tc(acc, xsin, deg2, bc, wq, bq, wk, bk, wn):
    has_next = wn is not None
    full = lambda i: (0, 0)
    in_specs = [
        pl.BlockSpec((2, _ROWS_B, D), lambda i: (0, i, 0)),
        pl.BlockSpec((_ROWS_B, D), lambda i: (i, 0)),
        pl.BlockSpec((2, _ROWS_B, DW), lambda i: (0, i, 0)),
        pl.BlockSpec((1, D), full),
        pl.BlockSpec((D, D), full),
        pl.BlockSpec((1, D), full),
        pl.BlockSpec((D, D), full),
        pl.BlockSpec((1, D), full),
    ]
    out_specs = [pl.BlockSpec((1, _GPB, D), lambda i: (i, 0, 0))]
    out_shape = [jax.ShapeDtypeStruct((G // _GPB, _GPB, D), jnp.float32)]
    args = [acc, xsin, deg2, bc.reshape(1, D), wq, bq.reshape(1, D), wk,
            bk.reshape(1, D)]
    if has_next:
        in_specs.append(pl.BlockSpec((D, D), full))
        out_specs.append(pl.BlockSpec((_ROWS_B, D), lambda i: (i, 0)))
        out_shape.append(jax.ShapeDtypeStruct((N, D), jnp.float32))
        args.append(wn)
    return pl.pallas_call(
        functools.partial(_layer_tc_body, has_next),
        grid=(N // _ROWS_B,),
        in_specs=in_specs,
        out_specs=out_specs,
        out_shape=out_shape,
    )(*args)


def kernel(x, edge_index, batch, Wc0, bc0, Wq0, bq0, Wk0, bk0,
           Wc1, bc1, Wq1, bq1, Wk1, bk1, Wc2, bc2, Wq2, bq2, Wk2, bk2):
    npd = EPAD - E
    pad2 = jnp.stack([jnp.arange(npd, dtype=jnp.int32) % 64,
                      N + (jnp.arange(npd, dtype=jnp.int32) % (NPAD - N))])
    eir = jnp.concatenate([edge_index.astype(jnp.int32), pad2], axis=1
                          ).reshape(2, 32, CHUNKS, CW)
    ones = jnp.ones((CW, DW), jnp.float32)
    z1 = jnp.zeros((640, DW), jnp.float32)

    deg2 = _deg_sc(eir, ones, z1)
    xs = _pre_tc(x, deg2, Wc0)
    params = [(bc0, Wq0, bq0, Wk0, bk0, Wc1),
              (bc1, Wq1, bq1, Wk1, bk1, Wc2),
              (bc2, Wq2, bq2, Wk2, bk2, None)]
    ros = []
    for bc, wq, bq, wk, bk, wn in params:
        acc = _scatter_sc(xs, eir)
        res = _layer_tc(acc, xs, deg2, bc, wq, bq, wk, bk, wn)
        if wn is not None:
            ro, xs = res
        else:
            (ro,) = res
        ros.append(ro.reshape(G, D))
    return jnp.concatenate(ros, axis=1)
